# baseline (speedup 1.0000x reference)
import functools
import math

import jax
import jax.numpy as jnp
from jax import lax
from jax.experimental import pallas as pl
from jax.experimental.pallas import tpu as pltpu
from jax.experimental.pallas import tpu_sc as plsc

_PADDING_IDX = 0
_SMOOTHING = 0.1
_CONFIDENCE = 1.0 - _SMOOTHING

_NC = 2
_NS = 16
_NW = _NC * _NS
_L = 16


def _sc_gather_body(n, v, eps, row_const, x_hbm, t_hbm, out_hbm, t_v, gi_v,
                    zi_v, gv_v, zv_v, acc_v, sem_t, sem_g, sem_z):
    rows_per_w = n // _NW
    chunks = rows_per_w // _L
    wid = lax.axis_index("s") * _NC + lax.axis_index("c")
    base = wid * rows_per_w

    pltpu.async_copy(t_hbm.at[pl.ds(base, rows_per_w)], t_v, sem_t).wait()

    for c in range(chunks):
        tv = t_v[pl.ds(c * _L, _L)]
        nv = (base + c * _L + lax.iota(jnp.int32, _L)) * v
        gi_v[pl.ds(c * _L, _L)] = nv + tv
        zi_v[pl.ds(c * _L, _L)] = nv

    dg = pltpu.async_copy(x_hbm.at[gi_v], gv_v, sem_g)
    dz = pltpu.async_copy(x_hbm.at[zi_v], zv_v, sem_z)
    dg.wait()
    dz.wait()

    acc = jnp.zeros((_L,), jnp.float32)
    for c in range(chunks):
        g = gv_v[pl.ds(c * _L, _L)]
        x0 = zv_v[pl.ds(c * _L, _L)]
        tv = t_v[pl.ds(c * _L, _L)]
        m = jnp.where(tv != _PADDING_IDX, 1.0, 0.0).astype(jnp.float32)
        acc = acc + m * ((eps - _CONFIDENCE) * g + eps * x0 + row_const)

    acc_v[...] = acc
    pltpu.sync_copy(acc_v, out_hbm.at[wid])


def _make_sc_gather(n, v, eps, row_const):
    rows_per_w = n // _NW
    body = functools.partial(_sc_gather_body, n, v, eps, row_const)
    return pl.kernel(
        body,
        mesh=plsc.VectorSubcoreMesh(core_axis_name="c", subcore_axis_name="s"),
        out_type=jax.ShapeDtypeStruct((_NW, _L), jnp.float32),
        scratch_types=[
            pltpu.VMEM((rows_per_w,), jnp.int32),
            pltpu.VMEM((rows_per_w,), jnp.int32),
            pltpu.VMEM((rows_per_w,), jnp.int32),
            pltpu.VMEM((rows_per_w,), jnp.float32),
            pltpu.VMEM((rows_per_w,), jnp.float32),
            pltpu.VMEM((_L,), jnp.float32),
            pltpu.SemaphoreType.DMA,
            pltpu.SemaphoreType.DMA,
            pltpu.SemaphoreType.DMA,
        ],
    )


def _red_kernel(t_ref, g_ref, x_ref, o_ref, *, eps):
    first = (pl.program_id(0) == 0) & (pl.program_id(1) == 0)

    x = x_ref[...]
    t = t_ref[0, 0, :]
    m = (t != _PADDING_IDX).astype(jnp.float32)
    bs = jnp.sum(x, axis=1)
    partial = -eps * jnp.sum(bs * m)

    @pl.when(first)
    def _():
        o_ref[...] = jnp.full((1, 1), jnp.sum(g_ref[...]), jnp.float32)

    o_ref[...] += jnp.full((1, 1), partial, dtype=jnp.float32)


def kernel(x, target):
    n, v = x.shape
    row_block = 512
    col_block = 3200
    nr = n // row_block
    nc = v // col_block

    eps = _SMOOTHING / (v - 2)
    row_const = _SMOOTHING * math.log(eps) + _CONFIDENCE * math.log(_CONFIDENCE)

    t32 = target.astype(jnp.int32)
    xflat = x.reshape(n * v)
    g = _make_sc_gather(n, v, eps, row_const)(xflat, t32)

    t3 = t32.reshape(nr, 1, row_block)

    out = pl.pallas_call(
        functools.partial(_red_kernel, eps=eps),
        grid=(nr, nc),
        in_specs=[
            pl.BlockSpec((1, 1, row_block), lambda i, j: (i, 0, 0)),
            pl.BlockSpec((_NW, _L), lambda i, j: (0, 0)),
            pl.BlockSpec((row_block, col_block), lambda i, j: (i, j)),
        ],
        out_specs=pl.BlockSpec((1, 1), lambda i, j: (0, 0)),
        out_shape=jax.ShapeDtypeStruct((1, 1), jnp.float32),
    )(t3, g, x)
    return out[0, 0]
